# Initial kernel scaffold; baseline (speedup 1.0000x reference)
import jax
import jax.numpy as jnp
from jax.experimental import pallas as pl


def kernel(precomputed_embeddings, chunk_bank, query_vectors, W1, b1, W2, b2):
    raise NotImplementedError("write your pallas kernel here")


# speedup vs baseline: 2.8392x; 2.8392x over previous
import functools

import jax
import jax.numpy as jnp
from jax import lax
from jax.experimental import pallas as pl
from jax.experimental.pallas import tpu as pltpu
from jax.experimental.pallas import tpu_sc as plsc

Q = 64
K = 32
D = 768
TEMP = 0.1
HID = 256
NCLS = 2
PEN_LAMBDA = 0.1
PEN_THRESH = 0.4

CHUNK_BLK = 4096
L = 128
RB = CHUNK_BLK // L
PRE_K = 6
QGRP = 16

NEG = float("-inf")
BIGI = 0x7FFFFFFF


def _fused_kernel(nblocks, n_valid, q_ref, c_ref, pen_ref, s_ref, cv_ref,
                  ci_ref, qn_s, cand_v, cand_i):
    b = pl.program_id(0)

    @pl.when(b == 0)
    def _():
        q = q_ref[...]
        n = jnp.sqrt(jnp.sum(q * q, axis=-1, keepdims=True))
        qn = q / jnp.maximum(n, 1e-12)
        qn_s[...] = qn
        sim = lax.dot_general(qn, qn, (((1,), (1,)), ((), ())))
        row = lax.broadcasted_iota(jnp.int32, (Q, Q), 0)
        col = lax.broadcasted_iota(jnp.int32, (Q, Q), 1)
        offdiag = (row != col).astype(jnp.float32)
        excess = jnp.maximum(sim - PEN_THRESH, 0.0) * offdiag
        pen_ref[...] = jnp.broadcast_to(
            PEN_LAMBDA * (jnp.sum(excess) / (Q * (Q - 1))), (1, 1)
        )
        cand_v[...] = jnp.full((Q, PRE_K, L), NEG, jnp.float32)
        cand_i[...] = jnp.zeros((Q, PRE_K, L), jnp.int32)

    c = c_ref[...]
    n = jnp.sqrt(jnp.sum(c * c, axis=-1, keepdims=True))
    cn = c / jnp.maximum(n, 1e-12)
    s = lax.dot_general(qn_s[...], cn, (((1,), (1,)), ((), ())))
    s_ref[...] = s

    s3 = s.reshape(Q, RB, L)
    base = b * CHUNK_BLK
    r_iota = lax.broadcasted_iota(jnp.int32, (Q, RB, L), 1)
    l_iota = lax.broadcasted_iota(jnp.int32, (Q, RB, L), 2)
    idx3 = base + r_iota * L + l_iota
    s3 = jnp.where(idx3 < n_valid, s3, NEG)

    kp = lax.broadcasted_iota(jnp.int32, (Q, PRE_K, L), 1)
    un_v = jnp.concatenate([cand_v[...], s3], axis=1)
    un_i = jnp.concatenate([cand_i[...], idx3], axis=1)

    def body(k, carry):
        un_v, acc_v, acc_i = carry
        m = jnp.max(un_v, axis=1, keepdims=True)
        wi = jnp.min(jnp.where(un_v == m, un_i, BIGI), axis=1, keepdims=True)
        un_v = jnp.where(un_i == wi, NEG, un_v)
        acc_v = jnp.where(kp == k, m, acc_v)
        acc_i = jnp.where(kp == k, wi, acc_i)
        return un_v, acc_v, acc_i

    acc_v = jnp.full((Q, PRE_K, L), NEG, jnp.float32)
    acc_i = jnp.zeros((Q, PRE_K, L), jnp.int32)
    _, acc_v, acc_i = lax.fori_loop(0, PRE_K, body, (un_v, acc_v, acc_i))
    cand_v[...] = acc_v
    cand_i[...] = acc_i

    @pl.when(b == nblocks - 1)
    def _():
        cv_ref[...] = acc_v
        ci_ref[...] = acc_i


def _fused_call(qv, chunk_bank, n_valid):
    nblocks = (n_valid + CHUNK_BLK - 1) // CHUNK_BLK
    n_pad = nblocks * CHUNK_BLK
    return pl.pallas_call(
        functools.partial(_fused_kernel, nblocks, n_valid),
        grid=(nblocks,),
        in_specs=[
            pl.BlockSpec((Q, D), lambda i: (0, 0)),
            pl.BlockSpec((CHUNK_BLK, D), lambda i: (i, 0)),
        ],
        out_specs=(
            pl.BlockSpec((1, 1), lambda i: (0, 0)),
            pl.BlockSpec((Q, CHUNK_BLK), lambda i: (0, i)),
            pl.BlockSpec((Q, PRE_K, L), lambda i: (0, 0, 0)),
            pl.BlockSpec((Q, PRE_K, L), lambda i: (0, 0, 0)),
        ),
        out_shape=(
            jax.ShapeDtypeStruct((1, 1), jnp.float32),
            jax.ShapeDtypeStruct((Q, n_pad), jnp.float32),
            jax.ShapeDtypeStruct((Q, PRE_K, L), jnp.float32),
            jax.ShapeDtypeStruct((Q, PRE_K, L), jnp.int32),
        ),
        scratch_shapes=[
            pltpu.VMEM((Q, D), jnp.float32),
            pltpu.VMEM((Q, PRE_K, L), jnp.float32),
            pltpu.VMEM((Q, PRE_K, L), jnp.int32),
        ],
    )(qv, chunk_bank)


def _sel_kernel(n_valid, cv_ref, ci_ref, s_hbm, ts_ref, ti_ref, fb_buf, sem):
    i = pl.program_id(0)
    lane32 = lax.broadcasted_iota(jnp.int32, (QGRP, K), 1)
    l_iota2 = lax.broadcasted_iota(jnp.int32, (QGRP, L), 1)

    def select32(vals, idxs, with_lanecnt):
        def body(k, carry):
            vals, accs, acci, lanecnt = carry
            m = jnp.max(jnp.max(vals, axis=2), axis=1)
            m3 = m[:, None, None]
            cand = jnp.where(vals == m3, idxs, BIGI)
            widx = jnp.min(jnp.min(cand, axis=2), axis=1)
            vals = jnp.where(idxs == widx[:, None, None], NEG, vals)
            accs = jnp.where(lane32 == k, m[:, None], accs)
            acci = jnp.where(lane32 == k, widx[:, None], acci)
            if with_lanecnt:
                lanecnt = lanecnt + jnp.where(
                    l_iota2 == (widx & (L - 1))[:, None], 1, 0
                )
            return vals, accs, acci, lanecnt

        accs = jnp.zeros((QGRP, K), jnp.float32)
        acci = jnp.zeros((QGRP, K), jnp.int32)
        lanecnt = jnp.zeros((QGRP, L), jnp.int32)
        _, accs, acci, lanecnt = lax.fori_loop(
            0, K, body, (vals, accs, acci, lanecnt)
        )
        return accs, acci, lanecnt

    accs, acci, lanecnt = select32(cv_ref[...], ci_ref[...], True)
    need_fb = jnp.max(lanecnt) >= PRE_K

    def fallback(_):
        cp = pltpu.make_async_copy(
            s_hbm.at[pl.ds(i * QGRP, QGRP), :], fb_buf, sem
        )
        cp.start()
        cp.wait()
        r = fb_buf.shape[1] // L
        r_iota = lax.broadcasted_iota(jnp.int32, (QGRP, r, L), 1)
        li = lax.broadcasted_iota(jnp.int32, (QGRP, r, L), 2)
        idx3 = r_iota * L + li
        vals = jnp.where(idx3 < n_valid, fb_buf[...].reshape(QGRP, r, L), NEG)
        a, b, _ = select32(vals, idx3, False)
        return a, b

    accs, acci = lax.cond(need_fb, fallback, lambda _: (accs, acci), 0)
    ts_ref[...] = accs
    ti_ref[...] = acci


def _sel_call(cand_v, cand_i, sims, n_valid):
    n_pad = sims.shape[1]
    return pl.pallas_call(
        functools.partial(_sel_kernel, n_valid),
        grid=(Q // QGRP,),
        in_specs=[
            pl.BlockSpec((QGRP, PRE_K, L), lambda i: (i, 0, 0)),
            pl.BlockSpec((QGRP, PRE_K, L), lambda i: (i, 0, 0)),
            pl.BlockSpec(memory_space=pl.ANY),
        ],
        out_specs=(
            pl.BlockSpec((QGRP, K), lambda i: (i, 0)),
            pl.BlockSpec((QGRP, K), lambda i: (i, 0)),
        ),
        out_shape=(
            jax.ShapeDtypeStruct((Q, K), jnp.float32),
            jax.ShapeDtypeStruct((Q, K), jnp.int32),
        ),
        scratch_shapes=[
            pltpu.VMEM((QGRP, n_pad), jnp.float32),
            pltpu.SemaphoreType.DMA,
        ],
    )(cand_v, cand_i, sims)


def _sc_gather(table, idx_flat):
    info = plsc.get_sparse_core_info()
    nw = info.num_cores * info.num_subcores
    b = idx_flat.shape[0]
    b_per_w = b // nw
    d = table.shape[1]
    mesh = plsc.VectorSubcoreMesh(core_axis_name="c", subcore_axis_name="s")

    @functools.partial(
        pl.kernel,
        mesh=mesh,
        out_type=jax.ShapeDtypeStruct((b, d), jnp.float32),
        scratch_types=[
            pltpu.VMEM((b_per_w,), jnp.int32),
            pltpu.VMEM((b_per_w, d), jnp.float32),
            pltpu.SemaphoreType.DMA,
        ],
    )
    def gather_k(table_hbm, idx_hbm, out_hbm, idx_v, rows_v, sem):
        wid = lax.axis_index("s") * info.num_cores + lax.axis_index("c")
        base = wid * b_per_w
        pltpu.sync_copy(idx_hbm.at[pl.ds(base, b_per_w)], idx_v)
        pltpu.async_copy(table_hbm.at[idx_v], rows_v, sem).wait()
        pltpu.sync_copy(rows_v, out_hbm.at[pl.ds(base, b_per_w)])

    return gather_k(table, idx_flat)


MLP_STEPS = 8
QB = Q // MLP_STEPS
MLP_BLK = QB * D


def _am_kernel(ts_ref, r_ref, w1_ref, b1_ref, w2_ref, b2_ref, out_ref,
               agg_s, acc):
    i = pl.program_id(0)

    @pl.when(i == 0)
    def _():
        ts = ts_ref[...] / TEMP
        m = jnp.max(ts, axis=-1, keepdims=True)
        e = jnp.exp(ts - m)
        attn = e / jnp.sum(e, axis=-1, keepdims=True)
        r3 = r_ref[...]
        n3 = jnp.sqrt(jnp.sum(r3 * r3, axis=-1, keepdims=True))
        rn3 = r3 / jnp.maximum(n3, 1e-12)
        agg_s[...] = lax.dot_general(attn, rn3, (((1,), (1,)), ((0,), (0,))))

    partial = jnp.zeros((1, HID), jnp.float32)
    for j in range(QB):
        a_row = agg_s[pl.ds(i * QB + j, 1), :]
        w_blk = w1_ref[pl.ds(j * D, D), :]
        partial = partial + jnp.dot(a_row, w_blk, preferred_element_type=jnp.float32)

    @pl.when(i == 0)
    def _():
        acc[...] = partial

    @pl.when(i > 0)
    def _():
        acc[...] = acc[...] + partial

    @pl.when(i == MLP_STEPS - 1)
    def _():
        h = jnp.maximum(acc[...] + b1_ref[...], 0.0)
        out_ref[...] = jnp.dot(h, w2_ref[...], preferred_element_type=jnp.float32) + b2_ref[...]


def _am_call(top_sims, retrieved, w1, b1, w2, b2):
    return pl.pallas_call(
        _am_kernel,
        grid=(MLP_STEPS,),
        in_specs=[
            pl.BlockSpec((Q, K), lambda i: (0, 0)),
            pl.BlockSpec((Q, K, D), lambda i: (0, 0, 0)),
            pl.BlockSpec((MLP_BLK, HID), lambda i: (i, 0)),
            pl.BlockSpec((1, HID), lambda i: (0, 0)),
            pl.BlockSpec((HID, NCLS), lambda i: (0, 0)),
            pl.BlockSpec((1, NCLS), lambda i: (0, 0)),
        ],
        out_specs=pl.BlockSpec((1, NCLS), lambda i: (0, 0)),
        out_shape=jax.ShapeDtypeStruct((1, NCLS), jnp.float32),
        scratch_shapes=[
            pltpu.VMEM((Q, D), jnp.float32),
            pltpu.VMEM((1, HID), jnp.float32),
        ],
    )(top_sims, retrieved, w1, b1, w2, b2)


def kernel(precomputed_embeddings, chunk_bank, query_vectors, W1, b1, W2, b2):
    batch = precomputed_embeddings.shape[0]
    n_chunks = chunk_bank.shape[0]

    pen, sims, cand_v, cand_i = _fused_call(query_vectors, chunk_bank, n_chunks)
    top_sims, top_idx = _sel_call(cand_v, cand_i, sims, n_chunks)
    retrieved = _sc_gather(chunk_bank, top_idx.reshape(-1))
    logits_row = _am_call(
        top_sims, retrieved.reshape(Q, K, D), W1,
        b1.reshape(1, HID), W2, b2.reshape(1, NCLS)
    )
    logits = jnp.broadcast_to(logits_row, (batch, NCLS))
    return logits, pen[0, 0], top_idx, top_sims


# speedup vs baseline: 3.4299x; 1.2080x over previous
import functools

import jax
import jax.numpy as jnp
from jax import lax
from jax.experimental import pallas as pl
from jax.experimental.pallas import tpu as pltpu
from jax.experimental.pallas import tpu_sc as plsc

Q = 64
K = 32
D = 768
TEMP = 0.1
HID = 256
NCLS = 2
PEN_LAMBDA = 0.1
PEN_THRESH = 0.4

CHUNK_BLK = 6144
L = 128
RB = CHUNK_BLK // L
PRE_K = 6

NEG = float("-inf")
BIGI = 0x7FFFFFFF


def _select32(vals, idxs, nq, with_lanecnt):
    lane32 = lax.broadcasted_iota(jnp.int32, (nq, K), 1)
    l_iota2 = lax.broadcasted_iota(jnp.int32, (nq, L), 1)

    def body(k, carry):
        vals, accs, acci, lanecnt = carry
        m = jnp.max(jnp.max(vals, axis=2), axis=1)
        m3 = m[:, None, None]
        cand = jnp.where(vals == m3, idxs, BIGI)
        widx = jnp.min(jnp.min(cand, axis=2), axis=1)
        vals = jnp.where(idxs == widx[:, None, None], NEG, vals)
        accs = jnp.where(lane32 == k, m[:, None], accs)
        acci = jnp.where(lane32 == k, widx[:, None], acci)
        if with_lanecnt:
            lanecnt = lanecnt + jnp.where(
                l_iota2 == (widx & (L - 1))[:, None], 1, 0
            )
        return vals, accs, acci, lanecnt

    accs = jnp.zeros((nq, K), jnp.float32)
    acci = jnp.zeros((nq, K), jnp.int32)
    lanecnt = jnp.zeros((nq, L), jnp.int32)
    _, accs, acci, lanecnt = lax.fori_loop(0, K, body, (vals, accs, acci, lanecnt))
    return accs, acci, lanecnt


def _fused_kernel(nblocks, n_valid, q_ref, c_ref, pen_ref, ts_ref, ti_ref,
                  fb_ref, qn_s, cand_v, cand_i):
    b = pl.program_id(0)

    @pl.when(b == 0)
    def _():
        q = q_ref[...]
        n = jnp.sqrt(jnp.sum(q * q, axis=-1, keepdims=True))
        qn = q / jnp.maximum(n, 1e-12)
        qn_s[...] = qn
        sim = lax.dot_general(qn, qn, (((1,), (1,)), ((), ())))
        row = lax.broadcasted_iota(jnp.int32, (Q, Q), 0)
        col = lax.broadcasted_iota(jnp.int32, (Q, Q), 1)
        offdiag = (row != col).astype(jnp.float32)
        excess = jnp.maximum(sim - PEN_THRESH, 0.0) * offdiag
        pen_ref[...] = jnp.broadcast_to(
            PEN_LAMBDA * (jnp.sum(excess) / (Q * (Q - 1))), (1, 1)
        )
        cand_v[...] = jnp.full((Q, PRE_K, L), NEG, jnp.float32)
        cand_i[...] = jnp.zeros((Q, PRE_K, L), jnp.int32)

    c = c_ref[...]
    n = jnp.sqrt(jnp.sum(c * c, axis=-1, keepdims=True))
    cn = c / jnp.maximum(n, 1e-12)
    s = lax.dot_general(qn_s[...], cn, (((1,), (1,)), ((), ())))

    s3 = s.reshape(Q, RB, L)
    base = b * CHUNK_BLK
    r_iota = lax.broadcasted_iota(jnp.int32, (Q, RB, L), 1)
    l_iota = lax.broadcasted_iota(jnp.int32, (Q, RB, L), 2)
    idx3 = base + r_iota * L + l_iota
    s3 = jnp.where(idx3 < n_valid, s3, NEG)

    kp = lax.broadcasted_iota(jnp.int32, (Q, PRE_K, L), 1)
    un_v = jnp.concatenate([cand_v[...], s3], axis=1)
    un_i = jnp.concatenate([cand_i[...], idx3], axis=1)

    def body(k, carry):
        un_v, acc_v, acc_i = carry
        m = jnp.max(un_v, axis=1, keepdims=True)
        wi = jnp.min(jnp.where(un_v == m, un_i, BIGI), axis=1, keepdims=True)
        un_v = jnp.where(un_i == wi, NEG, un_v)
        acc_v = jnp.where(kp == k, m, acc_v)
        acc_i = jnp.where(kp == k, wi, acc_i)
        return un_v, acc_v, acc_i

    acc_v = jnp.full((Q, PRE_K, L), NEG, jnp.float32)
    acc_i = jnp.zeros((Q, PRE_K, L), jnp.int32)
    _, acc_v, acc_i = lax.fori_loop(0, PRE_K, body, (un_v, acc_v, acc_i))
    cand_v[...] = acc_v
    cand_i[...] = acc_i

    @pl.when(b == nblocks - 1)
    def _():
        accs, acci, lanecnt = _select32(acc_v, acc_i, Q, True)
        ts_ref[...] = accs
        ti_ref[...] = acci
        fb_ref[...] = jnp.broadcast_to(
            (jnp.max(lanecnt) >= PRE_K).astype(jnp.int32), (1, 1)
        )


def _fused_call(qv, chunk_bank, n_valid):
    nblocks = (n_valid + CHUNK_BLK - 1) // CHUNK_BLK
    return pl.pallas_call(
        functools.partial(_fused_kernel, nblocks, n_valid),
        grid=(nblocks,),
        in_specs=[
            pl.BlockSpec((Q, D), lambda i: (0, 0)),
            pl.BlockSpec((CHUNK_BLK, D), lambda i: (i, 0)),
        ],
        out_specs=(
            pl.BlockSpec((1, 1), lambda i: (0, 0)),
            pl.BlockSpec((Q, K), lambda i: (0, 0)),
            pl.BlockSpec((Q, K), lambda i: (0, 0)),
            pl.BlockSpec((1, 1), lambda i: (0, 0)),
        ),
        out_shape=(
            jax.ShapeDtypeStruct((1, 1), jnp.float32),
            jax.ShapeDtypeStruct((Q, K), jnp.float32),
            jax.ShapeDtypeStruct((Q, K), jnp.int32),
            jax.ShapeDtypeStruct((1, 1), jnp.int32),
        ),
        scratch_shapes=[
            pltpu.VMEM((Q, D), jnp.float32),
            pltpu.VMEM((Q, PRE_K, L), jnp.float32),
            pltpu.VMEM((Q, PRE_K, L), jnp.int32),
        ],
    )(qv, chunk_bank)


def _fb_sims_kernel(q_ref, c_ref, s_ref, qn_s):
    b = pl.program_id(0)

    @pl.when(b == 0)
    def _():
        q = q_ref[...]
        n = jnp.sqrt(jnp.sum(q * q, axis=-1, keepdims=True))
        qn_s[...] = q / jnp.maximum(n, 1e-12)

    c = c_ref[...]
    n = jnp.sqrt(jnp.sum(c * c, axis=-1, keepdims=True))
    cn = c / jnp.maximum(n, 1e-12)
    s_ref[...] = lax.dot_general(qn_s[...], cn, (((1,), (1,)), ((), ())))


def _fb_sims_call(qv, chunk_bank, n_valid):
    nblocks = (n_valid + CHUNK_BLK - 1) // CHUNK_BLK
    n_pad = nblocks * CHUNK_BLK
    return pl.pallas_call(
        _fb_sims_kernel,
        grid=(nblocks,),
        in_specs=[
            pl.BlockSpec((Q, D), lambda i: (0, 0)),
            pl.BlockSpec((CHUNK_BLK, D), lambda i: (i, 0)),
        ],
        out_specs=pl.BlockSpec((Q, CHUNK_BLK), lambda i: (0, i)),
        out_shape=jax.ShapeDtypeStruct((Q, n_pad), jnp.float32),
        scratch_shapes=[pltpu.VMEM((Q, D), jnp.float32)],
    )(qv, chunk_bank)


FB_QGRP = 16


def _fb_topk_kernel(n_valid, s_ref, ts_ref, ti_ref):
    r = s_ref.shape[1]
    r_iota = lax.broadcasted_iota(jnp.int32, (FB_QGRP, r, L), 1)
    l_iota = lax.broadcasted_iota(jnp.int32, (FB_QGRP, r, L), 2)
    idx3 = r_iota * L + l_iota
    vals = jnp.where(idx3 < n_valid, s_ref[...], NEG)
    accs, acci, _ = _select32(vals, idx3, FB_QGRP, False)
    ts_ref[...] = accs
    ti_ref[...] = acci


def _fb_topk_call(sims3, n_valid):
    q, r, l = sims3.shape
    return pl.pallas_call(
        functools.partial(_fb_topk_kernel, n_valid),
        grid=(q // FB_QGRP,),
        in_specs=[pl.BlockSpec((FB_QGRP, r, l), lambda i: (i, 0, 0))],
        out_specs=(
            pl.BlockSpec((FB_QGRP, K), lambda i: (i, 0)),
            pl.BlockSpec((FB_QGRP, K), lambda i: (i, 0)),
        ),
        out_shape=(
            jax.ShapeDtypeStruct((q, K), jnp.float32),
            jax.ShapeDtypeStruct((q, K), jnp.int32),
        ),
    )(sims3)


def _sc_gather(table, idx_flat):
    info = plsc.get_sparse_core_info()
    nw = info.num_cores * info.num_subcores
    b = idx_flat.shape[0]
    b_per_w = b // nw
    d = table.shape[1]
    mesh = plsc.VectorSubcoreMesh(core_axis_name="c", subcore_axis_name="s")

    @functools.partial(
        pl.kernel,
        mesh=mesh,
        out_type=jax.ShapeDtypeStruct((b, d), jnp.float32),
        scratch_types=[
            pltpu.VMEM((b_per_w,), jnp.int32),
            pltpu.VMEM((b_per_w, d), jnp.float32),
            pltpu.SemaphoreType.DMA,
        ],
    )
    def gather_k(table_hbm, idx_hbm, out_hbm, idx_v, rows_v, sem):
        wid = lax.axis_index("s") * info.num_cores + lax.axis_index("c")
        base = wid * b_per_w
        pltpu.sync_copy(idx_hbm.at[pl.ds(base, b_per_w)], idx_v)
        pltpu.async_copy(table_hbm.at[idx_v], rows_v, sem).wait()
        pltpu.sync_copy(rows_v, out_hbm.at[pl.ds(base, b_per_w)])

    return gather_k(table, idx_flat)


MLP_STEPS = 8
QB = Q // MLP_STEPS
MLP_BLK = QB * D


def _am_kernel(ts_ref, r_ref, w1_ref, b1_ref, w2_ref, b2_ref, out_ref,
               agg_s, acc):
    i = pl.program_id(0)

    @pl.when(i == 0)
    def _():
        ts = ts_ref[...] / TEMP
        m = jnp.max(ts, axis=-1, keepdims=True)
        e = jnp.exp(ts - m)
        attn = e / jnp.sum(e, axis=-1, keepdims=True)
        r3 = r_ref[...]
        n3 = jnp.sqrt(jnp.sum(r3 * r3, axis=-1, keepdims=True))
        rn3 = r3 / jnp.maximum(n3, 1e-12)
        agg_s[...] = lax.dot_general(attn, rn3, (((1,), (1,)), ((0,), (0,))))

    partial = jnp.zeros((1, HID), jnp.float32)
    for j in range(QB):
        a_row = agg_s[pl.ds(i * QB + j, 1), :]
        w_blk = w1_ref[pl.ds(j * D, D), :]
        partial = partial + jnp.dot(a_row, w_blk, preferred_element_type=jnp.float32)

    @pl.when(i == 0)
    def _():
        acc[...] = partial

    @pl.when(i > 0)
    def _():
        acc[...] = acc[...] + partial

    @pl.when(i == MLP_STEPS - 1)
    def _():
        h = jnp.maximum(acc[...] + b1_ref[...], 0.0)
        out_ref[...] = jnp.dot(h, w2_ref[...], preferred_element_type=jnp.float32) + b2_ref[...]


def _am_call(top_sims, retrieved, w1, b1, w2, b2):
    return pl.pallas_call(
        _am_kernel,
        grid=(MLP_STEPS,),
        in_specs=[
            pl.BlockSpec((Q, K), lambda i: (0, 0)),
            pl.BlockSpec((Q, K, D), lambda i: (0, 0, 0)),
            pl.BlockSpec((MLP_BLK, HID), lambda i: (i, 0)),
            pl.BlockSpec((1, HID), lambda i: (0, 0)),
            pl.BlockSpec((HID, NCLS), lambda i: (0, 0)),
            pl.BlockSpec((1, NCLS), lambda i: (0, 0)),
        ],
        out_specs=pl.BlockSpec((1, NCLS), lambda i: (0, 0)),
        out_shape=jax.ShapeDtypeStruct((1, NCLS), jnp.float32),
        scratch_shapes=[
            pltpu.VMEM((Q, D), jnp.float32),
            pltpu.VMEM((1, HID), jnp.float32),
        ],
    )(top_sims, retrieved, w1, b1, w2, b2)


def kernel(precomputed_embeddings, chunk_bank, query_vectors, W1, b1, W2, b2):
    batch = precomputed_embeddings.shape[0]
    n_chunks = chunk_bank.shape[0]

    pen, ts, ti, fb = _fused_call(query_vectors, chunk_bank, n_chunks)

    def fb_path(ops):
        qv, cb = ops
        sims = _fb_sims_call(qv, cb, n_chunks)
        n_pad = sims.shape[1]
        return _fb_topk_call(sims.reshape(Q, n_pad // L, L), n_chunks)

    top_sims, top_idx = lax.cond(
        fb[0, 0] > 0,
        fb_path,
        lambda ops: (ts, ti),
        (query_vectors, chunk_bank),
    )

    retrieved = _sc_gather(chunk_bank, top_idx.reshape(-1))
    logits_row = _am_call(
        top_sims, retrieved.reshape(Q, K, D), W1,
        b1.reshape(1, HID), W2, b2.reshape(1, NCLS)
    )
    logits = jnp.broadcast_to(logits_row, (batch, NCLS))
    return logits, pen[0, 0], top_idx, top_sims
